# Initial kernel scaffold; baseline (speedup 1.0000x reference)
import jax
import jax.numpy as jnp
from jax.experimental import pallas as pl


def kernel(x, edge_index, W_root, W_nbr, b):
    raise NotImplementedError("write your pallas kernel here")


# speedup vs baseline: 5.5473x; 5.5473x over previous
import functools

import jax
import jax.numpy as jnp
from jax import lax
from jax.experimental import pallas as pl
from jax.experimental.pallas import tpu as pltpu
from jax.experimental.pallas import tpu_sc as plsc

N_NODES = 10000
N_EDGES = 320000
D = 128

NC = 2
NS = 16
NW = NC * NS

E_PER_TILE = N_EDGES // NW
E_CHK = 80
N_CHK = E_PER_TILE // E_CHK
N_PAD = 10240
ROWS_PER_TILE = N_PAD // NS

@functools.lru_cache(maxsize=1)
def _make_sc_aggregate():
    mesh = plsc.VectorSubcoreMesh(core_axis_name="c", subcore_axis_name="s")

    @functools.partial(
        pl.kernel,
        mesh=mesh,
        out_type=jax.ShapeDtypeStruct((NC * N_PAD, D), jnp.float32),
        scratch_types=[
            pltpu.VMEM((E_CHK,), jnp.int32),
            pltpu.VMEM((E_CHK,), jnp.int32),
            pltpu.VMEM((E_CHK, D), jnp.float32),
            pltpu.VMEM_SHARED((N_PAD, D), jnp.float32),
            pltpu.SemaphoreType.DMA,
        ],
    )
    def _sc_aggregate(src_hbm, dst_hbm, x_hbm, zeros_hbm, out_hbm,
                      sidx, didx, rows, acc, sem):
        c = lax.axis_index("c")
        s = lax.axis_index("s")
        tile = s * NC + c
        row0 = s * ROWS_PER_TILE

        pltpu.sync_copy(zeros_hbm, acc.at[pl.ds(row0, ROWS_PER_TILE)])
        plsc.subcore_barrier()

        edge0 = tile * E_PER_TILE

        def chunk(i, carry):
            base = edge0 + i * E_CHK
            pltpu.sync_copy(src_hbm.at[pl.ds(base, E_CHK)], sidx)
            pltpu.sync_copy(dst_hbm.at[pl.ds(base, E_CHK)], didx)
            pltpu.async_copy(x_hbm.at[sidx], rows, sem).wait()
            pltpu.sync_copy(rows, acc.at[didx], add=True)
            return carry

        lax.fori_loop(0, N_CHK, chunk, 0)

        plsc.subcore_barrier()
        pltpu.sync_copy(acc.at[pl.ds(row0, ROWS_PER_TILE)],
                        out_hbm.at[pl.ds(c * N_PAD + row0, ROWS_PER_TILE)])

    return _sc_aggregate


def _tc_body(x_ref, p0_ref, p1_ref, wr_ref, wn_ref, b_ref, o_ref):
    agg = p0_ref[...] + p1_ref[...]
    o = jnp.dot(x_ref[...], wr_ref[...], preferred_element_type=jnp.float32)
    o += jnp.dot(agg, wn_ref[...], preferred_element_type=jnp.float32)
    o += b_ref[...]
    o_ref[...] = jnp.maximum(o, 0.0)


_BLK = 1280
_NBLK = N_PAD // _BLK


def kernel(x, edge_index, W_root, W_nbr, b):
    src = edge_index[0].astype(jnp.int32)
    dst = edge_index[1].astype(jnp.int32)
    zeros = jnp.zeros((ROWS_PER_TILE, D), jnp.float32)

    partials = _make_sc_aggregate()(src, dst, x, zeros)

    out = pl.pallas_call(
        _tc_body,
        grid=(_NBLK,),
        in_specs=[
            pl.BlockSpec((_BLK, D), lambda i: (i, 0)),
            pl.BlockSpec((_BLK, D), lambda i: (i, 0)),
            pl.BlockSpec((_BLK, D), lambda i: (i + _NBLK, 0)),
            pl.BlockSpec((D, D), lambda i: (0, 0)),
            pl.BlockSpec((D, D), lambda i: (0, 0)),
            pl.BlockSpec((1, D), lambda i: (0, 0)),
        ],
        out_specs=pl.BlockSpec((_BLK, D), lambda i: (i, 0)),
        out_shape=jax.ShapeDtypeStruct((N_NODES, D), jnp.float32),
    )(x, partials, partials, W_root, W_nbr, b.reshape(1, D))
    return out
